# Initial kernel scaffold; baseline (speedup 1.0000x reference)
import jax
import jax.numpy as jnp
from jax.experimental import pallas as pl


def kernel(action, item_emb):
    raise NotImplementedError("write your pallas kernel here")


# speedup vs baseline: 1.6842x; 1.6842x over previous
import functools

import jax
import jax.numpy as jnp
from jax import lax
from jax.experimental import pallas as pl
from jax.experimental.pallas import tpu as pltpu
from jax.experimental.pallas import tpu_sc as plsc

NUM_ITEMS = 1000000
EMBED_DIM = 64
BATCH = 16384
HIST = 50

_INFO = plsc.get_sparse_core_info()
_NC, _NS = _INFO.num_cores, _INFO.num_subcores
_NW = _NC * _NS

_TOTAL = BATCH * HIST
_PER_W = _TOTAL // _NW
_GRP = 128
_NGRP = _PER_W // _GRP


def _sc_gather(idx2d, table, out_flat):
    mesh = plsc.VectorSubcoreMesh(core_axis_name="c", subcore_axis_name="s")

    @functools.partial(
        pl.kernel,
        out_type=jax.ShapeDtypeStruct((_TOTAL, EMBED_DIM), jnp.float32),
        mesh=mesh,
        scratch_types=[
            pltpu.VMEM((_NGRP, _GRP), jnp.int32),
            pltpu.VMEM((_GRP, EMBED_DIM), jnp.float32),
            pltpu.SemaphoreType.DMA,
        ],
        compiler_params=pltpu.CompilerParams(use_tc_tiling_on_sc=False),
    )
    def k(idx_hbm, table_hbm, out_hbm, idx_v, rows_v, gsem):
        wid = lax.axis_index("s") * _NC + lax.axis_index("c")
        pltpu.sync_copy(idx_hbm.at[pl.ds(wid * _NGRP, _NGRP)], idx_v)
        base = wid * _PER_W

        @pl.loop(0, _NGRP)
        def _(j):
            pltpu.async_copy(table_hbm.at[idx_v.at[j]], rows_v, gsem).wait()
            pltpu.sync_copy(rows_v, out_hbm.at[pl.ds(base + j * _GRP, _GRP)])

    return k(idx2d, table)


def kernel(action, item_emb):
    idx2d = action.reshape(_TOTAL // _GRP, _GRP).astype(jnp.int32)
    out = _sc_gather(idx2d, item_emb, None)
    return out.reshape(BATCH, HIST, EMBED_DIM)


# speedup vs baseline: 1.8725x; 1.1118x over previous
import functools

import jax
import jax.numpy as jnp
from jax import lax
from jax.experimental import pallas as pl
from jax.experimental.pallas import tpu as pltpu
from jax.experimental.pallas import tpu_sc as plsc

NUM_ITEMS = 1000000
EMBED_DIM = 64
BATCH = 16384
HIST = 50

_INFO = plsc.get_sparse_core_info()
_NC, _NS = _INFO.num_cores, _INFO.num_subcores
_NW = _NC * _NS

_TOTAL = BATCH * HIST
_PER_W = _TOTAL // _NW
_GRP = 128
_NGRP = _PER_W // _GRP


_NBUF = 8


def _sc_gather(idx2d, table, out_flat):
    mesh = plsc.VectorSubcoreMesh(core_axis_name="c", subcore_axis_name="s")

    @functools.partial(
        pl.kernel,
        out_type=jax.ShapeDtypeStruct((_TOTAL, EMBED_DIM), jnp.float32),
        mesh=mesh,
        scratch_types=[
            pltpu.VMEM((_NGRP, _GRP), jnp.int32),
            pltpu.VMEM((_NBUF, _GRP, EMBED_DIM), jnp.float32),
            pltpu.SemaphoreType.DMA((_NBUF,)),
            pltpu.SemaphoreType.DMA((_NBUF,)),
        ],
        compiler_params=pltpu.CompilerParams(use_tc_tiling_on_sc=False),
    )
    def k(idx_hbm, table_hbm, out_hbm, idx_v, rows_v, gsem, ssem):
        wid = lax.axis_index("s") * _NC + lax.axis_index("c")
        pltpu.sync_copy(idx_hbm.at[pl.ds(wid * _NGRP, _NGRP)], idx_v)
        base = wid * _PER_W

        def gather_start(g, b):
            pltpu.async_copy(table_hbm.at[idx_v.at[g]], rows_v.at[b],
                             gsem.at[b])

        def gather_wait(b):
            pltpu.make_async_copy(
                table_hbm.at[idx_v.at[0]], rows_v.at[b], gsem.at[b]).wait()

        def scatter_start(g, b):
            pltpu.async_copy(rows_v.at[b],
                             out_hbm.at[pl.ds(base + g * _GRP, _GRP)],
                             ssem.at[b])

        def scatter_wait(b):
            pltpu.make_async_copy(
                rows_v.at[b], out_hbm.at[pl.ds(base, _GRP)],
                ssem.at[b]).wait()

        for b in range(_NBUF):
            gather_start(b, b)

        @pl.loop(0, _NGRP - _NBUF, step=_NBUF)
        def _(s):
            for b in range(_NBUF):
                gather_wait(b)
                scatter_start(s + b, b)
            for b in range(_NBUF):
                scatter_wait(b)
                gather_start(s + b + _NBUF, b)

        for b in range(_NBUF):
            gather_wait(b)
            scatter_start(_NGRP - _NBUF + b, b)
        for b in range(_NBUF):
            scatter_wait(b)

    return k(idx2d, table)


def kernel(action, item_emb):
    idx2d = action.reshape(_TOTAL // _GRP, _GRP).astype(jnp.int32)
    out = _sc_gather(idx2d, item_emb, None)
    return out.reshape(BATCH, HIST, EMBED_DIM)
